# baseline (speedup 1.0000x reference)
import jax
import jax.numpy as jnp
from jax.experimental import pallas as pl
from jax.experimental.pallas import tpu as pltpu

_H = 64
_NL = 6
_CB = 1024
_B = 16
_L = 8192
_F32 = jnp.float32


def _dot(a, b):
    return jax.lax.dot_general(
        a, b, (((1,), (0,)), ((), ())),
        precision=jax.lax.Precision.HIGHEST,
        preferred_element_type=_F32)


_BF16 = jnp.bfloat16


def _dotb(a, b):
    return jax.lax.dot_general(
        a.astype(_BF16), b, (((1,), (0,)), ((), ())),
        preferred_element_type=_F32)


def _body(x_ref, cb_ref, cbt_ref, cbt16_ref, w0_ref, enclo_ref, enchi_ref, encb_ref,
          decw_ref, decb_ref, d5_ref, db5_ref,
          y_ref, ze_ref, ids_ref):
    x2 = x_ref[0].astype(_BF16).astype(_F32)
    ev = x2[:, 0:1]
    od = x2[:, 1:2]
    zr1 = jnp.zeros((1, 1), _F32)
    om1 = jnp.concatenate([zr1, od[:-1]], axis=0)
    ep1 = jnp.concatenate([ev[1:], zr1], axis=0)
    w0 = w0_ref[:].astype(_F32)
    z = (om1 * w0[0:1] + ev * w0[1:2] + od * w0[2:3] + ep1 * w0[3:4])
    z = jnp.maximum(z + encb_ref[0], 0.0)

    zr = jnp.zeros((1, _H), _F32)
    zrb = jnp.zeros((1, _H), _BF16)
    for i in range(1, _NL):
        t = _L >> i
        zb = z.astype(_BF16)
        zp = jnp.concatenate([zrb, zb, zrb], axis=0)
        r = zp.reshape(t // 2 + 1, 2, _H)
        ab = jnp.concatenate([r[:, 0, :], r[:, 1, :]], axis=1)
        acc = _dotb(ab[:-1], enclo_ref[i - 1])
        acc = acc + _dotb(ab[1:], enchi_ref[i - 1])
        z = jnp.maximum(acc + encb_ref[i], 0.0)

    cbt = cbt_ref[:]
    scores = _dotb(z, cbt16_ref[:])
    cbn = jnp.sum(cbt * cbt, axis=0, keepdims=True)
    z2 = jnp.sum(z * z, axis=1, keepdims=True)
    dist = z2 - 2.0 * scores + cbn
    dmin = jnp.min(dist, axis=1, keepdims=True)
    iota = jax.lax.broadcasted_iota(jnp.int32, (128, _CB), 1)
    ids = jnp.min(jnp.where(dist <= dmin, iota, _CB), axis=1)
    onehot = (iota == ids[:, None]).astype(_F32)
    e = _dot(onehot, cb_ref[:])

    ze_ref[0] = z
    ids_ref[0, 0, :] = ids.astype(jnp.int32)

    y = e
    for i in range(_NL - 1):
        t = 128 << i
        yb = y.astype(_BF16)
        xm1 = jnp.concatenate([zrb, yb[:-1]], axis=0)
        xp1 = jnp.concatenate([yb[1:], zrb], axis=0)
        ye = _dotb(yb, decw_ref[i, 1]) + _dotb(xm1, decw_ref[i, 3])
        yo = _dotb(yb, decw_ref[i, 2]) + _dotb(xp1, decw_ref[i, 0])
        bi = decb_ref[i]
        il = jnp.concatenate([(ye + bi)[:, None, :], (yo + bi)[:, None, :]],
                             axis=1)
        y = jnp.maximum(il.reshape(2 * t, _H), 0.0)

    yb = y.astype(_BF16)
    xm1 = jnp.concatenate([zrb, yb[:-1]], axis=0)
    xp1 = jnp.concatenate([yb[1:], zrb], axis=0)
    ye = _dotb(yb, d5_ref[1]) + _dotb(xm1, d5_ref[3])
    yo = _dotb(yb, d5_ref[2]) + _dotb(xp1, d5_ref[0])
    y_ref[0] = jnp.concatenate([ye, yo], axis=1) + db5_ref[:]


def kernel(x, codebook, enc_w0, enc_w1, enc_w2, enc_w3, enc_w4, enc_w5,
           enc_b0, enc_b1, enc_b2, enc_b3, enc_b4, enc_b5,
           dec_w0, dec_w1, dec_w2, dec_w3, dec_w4, dec_w5,
           dec_b0, dec_b1, dec_b2, dec_b3, dec_b4, dec_b5):
    enc_ws = [enc_w0, enc_w1, enc_w2, enc_w3, enc_w4, enc_w5]
    enc_bs = [enc_b0, enc_b1, enc_b2, enc_b3, enc_b4, enc_b5]
    dec_ws = [dec_w0, dec_w1, dec_w2, dec_w3, dec_w4, dec_w5]
    dec_bs = [dec_b0, dec_b1, dec_b2, dec_b3, dec_b4, dec_b5]

    w0 = enc_ws[0][:, 0, :].T
    enclo = jnp.stack(
        [jnp.concatenate([enc_ws[i][:, :, 0].T, enc_ws[i][:, :, 1].T], axis=0)
         for i in range(1, _NL)])
    enchi = jnp.stack(
        [jnp.concatenate([enc_ws[i][:, :, 2].T, enc_ws[i][:, :, 3].T], axis=0)
         for i in range(1, _NL)])
    encb = jnp.stack([b[None, :] for b in enc_bs])
    decw = jnp.stack([jnp.stack([dec_ws[i][:, :, k] for k in range(4)])
                      for i in range(_NL - 1)])
    decb = jnp.stack([b[None, :] for b in dec_bs[:-1]])
    d5 = jnp.stack([dec_ws[-1][:, :, k] for k in range(4)])
    db5 = dec_bs[-1][None, :]
    cbt = codebook.T
    cbt16 = cbt.astype(jnp.bfloat16)
    w0 = w0.astype(jnp.bfloat16)
    enclo = enclo.astype(jnp.bfloat16)
    enchi = enchi.astype(jnp.bfloat16)
    decw = decw.astype(jnp.bfloat16)
    d5 = d5.astype(jnp.bfloat16)
    xp = x.reshape(_B, _L // 2, 2)

    def full(shape):
        nd = len(shape)
        return pl.BlockSpec(shape, lambda b, _n=nd: (0,) * _n)

    y, ze, ids = pl.pallas_call(
        _body,
        grid=(_B,),
        in_specs=[
            pl.BlockSpec((1, _L // 2, 2), lambda b: (b, 0, 0)),
            full((_CB, _H)), full((_H, _CB)), full((_H, _CB)), full((4, _H)),
            full((5, 2 * _H, _H)), full((5, 2 * _H, _H)), full((6, 1, _H)),
            full((5, 4, _H, _H)), full((5, 1, _H)),
            full((4, _H, 1)), full((1, 1)),
        ],
        out_specs=[
            pl.BlockSpec((1, _L // 2, 2), lambda b: (b, 0, 0)),
            pl.BlockSpec((1, 128, _H), lambda b: (b, 0, 0)),
            pl.BlockSpec((1, 1, 128), lambda b: (b, 0, 0)),
        ],
        out_shape=[
            jax.ShapeDtypeStruct((_B, _L // 2, 2), _F32),
            jax.ShapeDtypeStruct((_B, 128, _H), _F32),
            jax.ShapeDtypeStruct((_B, 1, 128), jnp.int32),
        ],
        compiler_params=pltpu.CompilerParams(
            dimension_semantics=("parallel",)),
    )(xp, codebook, cbt, cbt16, w0, enclo, enchi, encb, decw, decb, d5, db5)

    return (y.reshape(_B, 1, _L), ze.reshape(_B * 128, _H),
            ids.reshape(_B * 128))


# speedup vs baseline: 2.2835x; 2.2835x over previous
import jax
import jax.numpy as jnp
import numpy as np
from jax.experimental import pallas as pl
from jax.experimental.pallas import tpu as pltpu

_H = 64
_NL = 6
_CB = 1024
_B = 16
_L = 8192
_F32 = jnp.float32
_BF16 = jnp.bfloat16


def _dotb(a, b):
    return jax.lax.dot_general(
        a, b, (((1,), (0,)), ((), ())),
        preferred_element_type=_F32)


def _sd(a):
    return jnp.concatenate([jnp.zeros((1, a.shape[1]), a.dtype), a[:-1]], axis=0)


def _su(a):
    return jnp.concatenate([a[1:], jnp.zeros((1, a.shape[1]), a.dtype)], axis=0)


def _wrap(z):
    return jnp.concatenate([_sd(z[:, -_H:]), z, _su(z[:, :_H])], axis=1)


def _body(x_ref, cb_ref, cb16_ref, cbt_ref, cbt16_ref, w0_ref,
          enclo_ref, enchi_ref, encb_ref, decw_ref, decb_ref,
          d5_ref, db5_ref, y_ref, ze_ref, ids_ref):
    x0 = x_ref[0].astype(_BF16)
    x0x = jnp.concatenate([_sd(x0[:, -1:]), x0, _su(x0[:, :1])], axis=1)
    z = _dotb(x0x, w0_ref[:])
    b0 = encb_ref[0]
    z = jnp.maximum(z + jnp.concatenate([b0] * 32, axis=1), 0.0)

    for i in range(1, _NL):
        p = 1 << (6 - i)
        zx = _wrap(z.astype(_BF16))
        outs = []
        for q in range(p // 2):
            lo = zx[:, 128 * q:128 * q + 128]
            hi = zx[:, 128 * q + 128:128 * q + 256]
            outs.append(_dotb(lo, enclo_ref[i - 1])
                        + _dotb(hi, enchi_ref[i - 1]))
        acc = outs[0] if len(outs) == 1 else jnp.concatenate(outs, axis=1)
        bt = (encb_ref[i] if p == 2
              else jnp.concatenate([encb_ref[i]] * (p // 2), axis=1))
        z = jnp.maximum(acc + bt, 0.0)

    cbt = cbt_ref[:]
    scores = _dotb(z.astype(_BF16), cbt16_ref[:])
    cbn = jnp.sum(cbt * cbt, axis=0, keepdims=True)
    z2 = jnp.sum(z * z, axis=1, keepdims=True)
    dist = z2 - 2.0 * scores + cbn
    dmin = jnp.min(dist, axis=1, keepdims=True)
    iota = jax.lax.broadcasted_iota(jnp.int32, (128, _CB), 1)
    ids = jnp.min(jnp.where(dist <= dmin, iota, _CB), axis=1)
    onehot = (iota == ids[:, None]).astype(_BF16)
    e = _dotb(onehot, cb16_ref[:])

    ze_ref[0] = z
    ids_ref[0, 0, :] = ids.astype(jnp.int32)

    y = e
    for i in range(_NL - 1):
        p = 1 << i
        yx = _wrap(y.astype(_BF16))
        outs = []
        for q in range(p):
            xm = yx[:, 64 * q:64 * q + 64]
            xc = yx[:, 64 * q + 64:64 * q + 128]
            xq = yx[:, 64 * q + 128:64 * q + 192]
            outs.append(_dotb(xc, decw_ref[i, 1]) + _dotb(xm, decw_ref[i, 3]))
            outs.append(_dotb(xc, decw_ref[i, 2]) + _dotb(xq, decw_ref[i, 0]))
        acc = jnp.concatenate(outs, axis=1)
        bt = jnp.concatenate([decb_ref[i]] * (2 * p), axis=1)
        y = jnp.maximum(acc + bt, 0.0)

    y5 = _wrap(y.astype(_BF16))
    yout = _dotb(y5, d5_ref[:]) + db5_ref[:]
    y_ref[0] = yout


def kernel(x, codebook, enc_w0, enc_w1, enc_w2, enc_w3, enc_w4, enc_w5,
           enc_b0, enc_b1, enc_b2, enc_b3, enc_b4, enc_b5,
           dec_w0, dec_w1, dec_w2, dec_w3, dec_w4, dec_w5,
           dec_b0, dec_b1, dec_b2, dec_b3, dec_b4, dec_b5):
    enc_ws = [enc_w0, enc_w1, enc_w2, enc_w3, enc_w4, enc_w5]
    enc_bs = [enc_b0, enc_b1, enc_b2, enc_b3, enc_b4, enc_b5]
    dec_ws = [dec_w0, dec_w1, dec_w2, dec_w3, dec_w4, dec_w5]
    dec_bs = [dec_b0, dec_b1, dec_b2, dec_b3, dec_b4, dec_b5]

    w0t = enc_ws[0][:, 0, :].T
    qs = np.repeat(np.arange(32), 4)
    ks = np.tile(np.arange(4), 32)
    w0b = jnp.zeros((66, 32, _H), _F32)
    w0b = w0b.at[2 * qs + ks, qs, :].set(jnp.tile(w0t, (32, 1)))
    w0b = w0b.reshape(66, 32 * _H).astype(_BF16)

    enclo = jnp.stack(
        [jnp.concatenate([enc_ws[i][:, :, 0].T, enc_ws[i][:, :, 1].T], axis=0)
         for i in range(1, _NL)]).astype(_BF16)
    enchi = jnp.stack(
        [jnp.concatenate([enc_ws[i][:, :, 2].T, enc_ws[i][:, :, 3].T], axis=0)
         for i in range(1, _NL)]).astype(_BF16)
    encb = jnp.stack([b[None, :] for b in enc_bs])
    decw = jnp.stack([jnp.stack([dec_ws[i][:, :, k] for k in range(4)])
                      for i in range(_NL - 1)]).astype(_BF16)
    decb = jnp.stack([b[None, :] for b in dec_bs[:-1]])

    d5n = dec_ws[-1][:, 0, :]
    ps = np.arange(32)
    w5 = jnp.zeros((34, _H, _H), _F32)
    for j_off, s_off, k_ in ((1, 0, 1), (0, 0, 3), (1, 1, 2), (2, 1, 0)):
        w5 = w5.at[ps + j_off, :, 2 * ps + s_off].add(
            jnp.broadcast_to(d5n[:, k_], (32, _H)))
    d5 = w5.reshape(34 * _H, _H).astype(_BF16)
    db5 = dec_bs[-1][None, :]

    cbt = codebook.T
    cbt16 = cbt.astype(_BF16)
    cb16 = codebook.astype(_BF16)
    xp = x.reshape(_B, 128, _H)

    def full(shape):
        nd = len(shape)
        return pl.BlockSpec(shape, lambda b, _n=nd: (0,) * _n)

    y, ze, ids = pl.pallas_call(
        _body,
        grid=(_B,),
        in_specs=[
            pl.BlockSpec((1, 128, _H), lambda b: (b, 0, 0)),
            full((_CB, _H)), full((_CB, _H)), full((_H, _CB)), full((_H, _CB)),
            full((66, 32 * _H)),
            full((5, 2 * _H, _H)), full((5, 2 * _H, _H)), full((6, 1, _H)),
            full((5, 4, _H, _H)), full((5, 1, _H)),
            full((34 * _H, _H)), full((1, 1)),
        ],
        out_specs=[
            pl.BlockSpec((1, 128, _H), lambda b: (b, 0, 0)),
            pl.BlockSpec((1, 128, _H), lambda b: (b, 0, 0)),
            pl.BlockSpec((1, 1, 128), lambda b: (b, 0, 0)),
        ],
        out_shape=[
            jax.ShapeDtypeStruct((_B, 128, _H), _F32),
            jax.ShapeDtypeStruct((_B, 128, _H), _F32),
            jax.ShapeDtypeStruct((_B, 1, 128), jnp.int32),
        ],
        compiler_params=pltpu.CompilerParams(
            dimension_semantics=("parallel",)),
    )(xp, codebook, cb16, cbt, cbt16, w0b, enclo, enchi, encb,
      decw, decb, d5, db5)

    return (y.reshape(_B, 1, _L), ze.reshape(_B * 128, _H),
            ids.reshape(_B * 128))


# speedup vs baseline: 2.5645x; 1.1230x over previous
import jax
import jax.numpy as jnp
import numpy as np
from jax.experimental import pallas as pl
from jax.experimental.pallas import tpu as pltpu

_H = 64
_NL = 6
_CB = 1024
_B = 16
_L = 8192
_F32 = jnp.float32
_BF16 = jnp.bfloat16


def _dotb(a, b):
    return jax.lax.dot_general(
        a, b, (((1,), (0,)), ((), ())),
        preferred_element_type=_F32)


def _sd(a):
    return jnp.concatenate([jnp.zeros((1, a.shape[1]), a.dtype), a[:-1]], axis=0)


def _su(a):
    return jnp.concatenate([a[1:], jnp.zeros((1, a.shape[1]), a.dtype)], axis=0)


def _wrap(z):
    return jnp.concatenate([_sd(z[:, -_H:]), z, _su(z[:, :_H])], axis=1)


def _body(x_ref, cb_ref, cb16_ref, cbt_ref, cbt16_ref, w0_ref,
          enclo_ref, enchi_ref, encb_ref, decw_ref, decb_ref,
          d5_ref, db5_ref, y_ref, ze_ref, ids_ref):
    x0 = x_ref[0].astype(_BF16)
    x0x = jnp.concatenate([_sd(x0[:, -1:]), x0, _su(x0[:, :1])], axis=1)
    z = _dotb(x0x, w0_ref[:])
    b0 = encb_ref[0]
    z = jnp.maximum(z + jnp.concatenate([b0] * 32, axis=1), 0.0)

    for i in range(1, _NL):
        p = 1 << (6 - i)
        zx = _wrap(z.astype(_BF16))
        outs = []
        for q in range(p // 2):
            lo = zx[:, 128 * q:128 * q + 128]
            hi = zx[:, 128 * q + 128:128 * q + 256]
            outs.append(_dotb(lo, enclo_ref[i - 1])
                        + _dotb(hi, enchi_ref[i - 1]))
        acc = outs[0] if len(outs) == 1 else jnp.concatenate(outs, axis=1)
        bt = (encb_ref[i] if p == 2
              else jnp.concatenate([encb_ref[i]] * (p // 2), axis=1))
        z = jnp.maximum(acc + bt, 0.0)

    cbt = cbt_ref[:]
    scores = _dotb(z.astype(_BF16), cbt16_ref[:])
    cbn = jnp.sum(cbt * cbt, axis=0, keepdims=True)
    z2 = jnp.sum(z * z, axis=1, keepdims=True)
    dist = z2 - 2.0 * scores + cbn
    dmin = jnp.min(dist, axis=1, keepdims=True)
    iota = jax.lax.broadcasted_iota(jnp.int32, (128, _CB), 1)
    ids = jnp.min(jnp.where(dist <= dmin, iota, _CB), axis=1)
    onehot = (iota == ids[:, None]).astype(_BF16)
    e = _dotb(onehot, cb16_ref[:])

    ze_ref[0] = z
    ids_ref[0, 0, :] = ids.astype(jnp.int32)

    y = e
    for i in range(_NL - 1):
        p = 1 << i
        yx = _wrap(y.astype(_BF16))
        outs = []
        for q in range(p):
            xm = yx[:, 64 * q:64 * q + 64]
            xc = yx[:, 64 * q + 64:64 * q + 128]
            xq = yx[:, 64 * q + 128:64 * q + 192]
            outs.append(_dotb(xc, decw_ref[i, 1]) + _dotb(xm, decw_ref[i, 3]))
            outs.append(_dotb(xc, decw_ref[i, 2]) + _dotb(xq, decw_ref[i, 0]))
        acc = jnp.concatenate(outs, axis=1)
        bt = jnp.concatenate([decb_ref[i]] * (2 * p), axis=1)
        y = jnp.maximum(acc + bt, 0.0)

    y5 = _wrap(y.astype(_BF16))
    yout = _dotb(y5, d5_ref[:]) + db5_ref[:]
    y_ref[0] = yout


def kernel(x, codebook, enc_w0, enc_w1, enc_w2, enc_w3, enc_w4, enc_w5,
           enc_b0, enc_b1, enc_b2, enc_b3, enc_b4, enc_b5,
           dec_w0, dec_w1, dec_w2, dec_w3, dec_w4, dec_w5,
           dec_b0, dec_b1, dec_b2, dec_b3, dec_b4, dec_b5):
    enc_ws = [enc_w0, enc_w1, enc_w2, enc_w3, enc_w4, enc_w5]
    enc_bs = [enc_b0, enc_b1, enc_b2, enc_b3, enc_b4, enc_b5]
    dec_ws = [dec_w0, dec_w1, dec_w2, dec_w3, dec_w4, dec_w5]
    dec_bs = [dec_b0, dec_b1, dec_b2, dec_b3, dec_b4, dec_b5]

    w0t = enc_ws[0][:, 0, :].T
    pat0 = np.zeros((4, 66, 32, 1), np.float32)
    for q_ in range(32):
        for k_ in range(4):
            pat0[k_, 2 * q_ + k_, q_, 0] = 1.0
    w0b = sum(pat0[k_] * w0t[k_][None, None, :] for k_ in range(4))
    w0b = w0b.reshape(66, 32 * _H).astype(_BF16)

    enclo = jnp.stack(
        [jnp.concatenate([enc_ws[i][:, :, 0].T, enc_ws[i][:, :, 1].T], axis=0)
         for i in range(1, _NL)]).astype(_BF16)
    enchi = jnp.stack(
        [jnp.concatenate([enc_ws[i][:, :, 2].T, enc_ws[i][:, :, 3].T], axis=0)
         for i in range(1, _NL)]).astype(_BF16)
    encb = jnp.stack([b[None, :] for b in enc_bs])
    decw = jnp.stack([jnp.stack([dec_ws[i][:, :, k] for k in range(4)])
                      for i in range(_NL - 1)]).astype(_BF16)
    decb = jnp.stack([b[None, :] for b in dec_bs[:-1]])

    d5n = dec_ws[-1][:, 0, :]
    terms = ((1, 0, 1), (0, 0, 3), (1, 1, 2), (2, 1, 0))
    pat5 = np.zeros((4, 34, 1, _H), np.float32)
    for t_, (j_off, s_off, k_) in enumerate(terms):
        for p_ in range(32):
            pat5[t_, p_ + j_off, 0, 2 * p_ + s_off] = 1.0
    w5 = sum(pat5[t_] * d5n[:, k_][None, :, None]
             for t_, (j_off, s_off, k_) in enumerate(terms))
    d5 = w5.reshape(34 * _H, _H).astype(_BF16)
    db5 = dec_bs[-1][None, :]

    cbt = codebook.T
    cbt16 = cbt.astype(_BF16)
    cb16 = codebook.astype(_BF16)
    xp = x.reshape(_B, 128, _H)

    def full(shape):
        nd = len(shape)
        return pl.BlockSpec(shape, lambda b, _n=nd: (0,) * _n)

    y, ze, ids = pl.pallas_call(
        _body,
        grid=(_B,),
        in_specs=[
            pl.BlockSpec((1, 128, _H), lambda b: (b, 0, 0)),
            full((_CB, _H)), full((_CB, _H)), full((_H, _CB)), full((_H, _CB)),
            full((66, 32 * _H)),
            full((5, 2 * _H, _H)), full((5, 2 * _H, _H)), full((6, 1, _H)),
            full((5, 4, _H, _H)), full((5, 1, _H)),
            full((34 * _H, _H)), full((1, 1)),
        ],
        out_specs=[
            pl.BlockSpec((1, 128, _H), lambda b: (b, 0, 0)),
            pl.BlockSpec((1, 128, _H), lambda b: (b, 0, 0)),
            pl.BlockSpec((1, 1, 128), lambda b: (b, 0, 0)),
        ],
        out_shape=[
            jax.ShapeDtypeStruct((_B, 128, _H), _F32),
            jax.ShapeDtypeStruct((_B, 128, _H), _F32),
            jax.ShapeDtypeStruct((_B, 1, 128), jnp.int32),
        ],
        compiler_params=pltpu.CompilerParams(
            dimension_semantics=("parallel",)),
    )(xp, codebook, cb16, cbt, cbt16, w0b, enclo, enchi, encb,
      decw, decb, d5, db5)

    return (y.reshape(_B, 1, _L), ze.reshape(_B * 128, _H),
            ids.reshape(_B * 128))


# speedup vs baseline: 4.1667x; 1.6248x over previous
import jax
import jax.numpy as jnp
import numpy as np
from jax.experimental import pallas as pl
from jax.experimental.pallas import tpu as pltpu

_H = 64
_NL = 6
_CB = 1024
_B = 16
_L = 8192
_F32 = jnp.float32
_BF16 = jnp.bfloat16


def _dotb(a, b):
    return jax.lax.dot_general(
        a, b, (((1,), (0,)), ((), ())),
        preferred_element_type=_F32)


_NB = 4
_R = 128 * _NB


def _row_iota():
    return jax.lax.broadcasted_iota(jnp.int32, (_R, 1), 0)


def _sd(a):
    s = jnp.concatenate([jnp.zeros((1, a.shape[1]), a.dtype), a[:-1]], axis=0)
    return jnp.where((_row_iota() & 127) != 0, s, jnp.zeros_like(s))


def _su(a):
    s = jnp.concatenate([a[1:], jnp.zeros((1, a.shape[1]), a.dtype)], axis=0)
    return jnp.where((_row_iota() & 127) != 127, s, jnp.zeros_like(s))


def _wrap(z):
    return jnp.concatenate([_sd(z[:, -_H:]), z, _su(z[:, :_H])], axis=1)


def _body(x_ref, cb_ref, cb16_ref, cbt_ref, cbt16_ref, w0_ref,
          enclo_ref, enchi_ref, encb_ref, decw_ref, decb_ref,
          d5_ref, db5_ref, y_ref, ze_ref, ids_ref):
    x0 = x_ref[:].reshape(_R, _H).astype(_BF16)
    x0x = jnp.concatenate([_sd(x0[:, -1:]), x0, _su(x0[:, :1])], axis=1)
    z = _dotb(x0x, w0_ref[:])
    b0 = encb_ref[0]
    z = jnp.maximum(z + jnp.concatenate([b0] * 32, axis=1), 0.0)

    for i in range(1, _NL):
        p = 1 << (6 - i)
        zx = _wrap(z.astype(_BF16))
        outs = []
        for q in range(p // 2):
            lo = zx[:, 128 * q:128 * q + 128]
            hi = zx[:, 128 * q + 128:128 * q + 256]
            outs.append(_dotb(lo, enclo_ref[i - 1])
                        + _dotb(hi, enchi_ref[i - 1]))
        acc = outs[0] if len(outs) == 1 else jnp.concatenate(outs, axis=1)
        bt = (encb_ref[i] if p == 2
              else jnp.concatenate([encb_ref[i]] * (p // 2), axis=1))
        z = jnp.maximum(acc + bt, 0.0)

    cbt = cbt_ref[:]
    scores = _dotb(z.astype(_BF16), cbt16_ref[:])
    cbn = jnp.sum(cbt * cbt, axis=0, keepdims=True)
    z2 = jnp.sum(z * z, axis=1, keepdims=True)
    dist = z2 - 2.0 * scores + cbn
    dmin = jnp.min(dist, axis=1, keepdims=True)
    iota = jax.lax.broadcasted_iota(jnp.int32, (_R, _CB), 1)
    ids = jnp.min(jnp.where(dist <= dmin, iota, _CB), axis=1)
    onehot = (iota == ids[:, None]).astype(_BF16)
    e = _dotb(onehot, cb16_ref[:])

    ze_ref[:] = z.reshape(_NB, 128, _H)
    ids_ref[0] = ids.astype(jnp.int32).reshape(_NB, 128)

    y = e
    for i in range(_NL - 1):
        p = 1 << i
        yx = _wrap(y.astype(_BF16))
        outs = []
        for q in range(p):
            xm = yx[:, 64 * q:64 * q + 64]
            xc = yx[:, 64 * q + 64:64 * q + 128]
            xq = yx[:, 64 * q + 128:64 * q + 192]
            outs.append(_dotb(xc, decw_ref[i, 1]) + _dotb(xm, decw_ref[i, 3]))
            outs.append(_dotb(xc, decw_ref[i, 2]) + _dotb(xq, decw_ref[i, 0]))
        acc = jnp.concatenate(outs, axis=1)
        bt = jnp.concatenate([decb_ref[i]] * (2 * p), axis=1)
        y = jnp.maximum(acc + bt, 0.0)

    y5 = _wrap(y.astype(_BF16))
    yout = _dotb(y5, d5_ref[:]) + db5_ref[:]
    y_ref[:] = yout.reshape(_NB, 128, _H)


def kernel(x, codebook, enc_w0, enc_w1, enc_w2, enc_w3, enc_w4, enc_w5,
           enc_b0, enc_b1, enc_b2, enc_b3, enc_b4, enc_b5,
           dec_w0, dec_w1, dec_w2, dec_w3, dec_w4, dec_w5,
           dec_b0, dec_b1, dec_b2, dec_b3, dec_b4, dec_b5):
    enc_ws = [enc_w0, enc_w1, enc_w2, enc_w3, enc_w4, enc_w5]
    enc_bs = [enc_b0, enc_b1, enc_b2, enc_b3, enc_b4, enc_b5]
    dec_ws = [dec_w0, dec_w1, dec_w2, dec_w3, dec_w4, dec_w5]
    dec_bs = [dec_b0, dec_b1, dec_b2, dec_b3, dec_b4, dec_b5]

    w0t = enc_ws[0][:, 0, :].T
    pat0 = np.zeros((4, 66, 32, 1), np.float32)
    for q_ in range(32):
        for k_ in range(4):
            pat0[k_, 2 * q_ + k_, q_, 0] = 1.0
    w0b = sum(pat0[k_] * w0t[k_][None, None, :] for k_ in range(4))
    w0b = w0b.reshape(66, 32 * _H).astype(_BF16)

    enclo = jnp.stack(
        [jnp.concatenate([enc_ws[i][:, :, 0].T, enc_ws[i][:, :, 1].T], axis=0)
         for i in range(1, _NL)]).astype(_BF16)
    enchi = jnp.stack(
        [jnp.concatenate([enc_ws[i][:, :, 2].T, enc_ws[i][:, :, 3].T], axis=0)
         for i in range(1, _NL)]).astype(_BF16)
    encb = jnp.stack([b[None, :] for b in enc_bs])
    decw = jnp.stack([jnp.stack([dec_ws[i][:, :, k] for k in range(4)])
                      for i in range(_NL - 1)]).astype(_BF16)
    decb = jnp.stack([b[None, :] for b in dec_bs[:-1]])

    d5n = dec_ws[-1][:, 0, :]
    terms = ((1, 0, 1), (0, 0, 3), (1, 1, 2), (2, 1, 0))
    pat5 = np.zeros((4, 34, 1, _H), np.float32)
    for t_, (j_off, s_off, k_) in enumerate(terms):
        for p_ in range(32):
            pat5[t_, p_ + j_off, 0, 2 * p_ + s_off] = 1.0
    w5 = sum(pat5[t_] * d5n[:, k_][None, :, None]
             for t_, (j_off, s_off, k_) in enumerate(terms))
    d5 = w5.reshape(34 * _H, _H).astype(_BF16)
    db5 = dec_bs[-1][None, :]

    cbt = codebook.T
    cbt16 = cbt.astype(_BF16)
    cb16 = codebook.astype(_BF16)
    xp = x.reshape(_B, 128, _H)

    def full(shape):
        nd = len(shape)
        return pl.BlockSpec(shape, lambda b, _n=nd: (0,) * _n)

    y, ze, ids = pl.pallas_call(
        _body,
        grid=(_B // _NB,),
        in_specs=[
            pl.BlockSpec((_NB, 128, _H), lambda b: (b, 0, 0)),
            full((_CB, _H)), full((_CB, _H)), full((_H, _CB)), full((_H, _CB)),
            full((66, 32 * _H)),
            full((5, 2 * _H, _H)), full((5, 2 * _H, _H)), full((6, 1, _H)),
            full((5, 4, _H, _H)), full((5, 1, _H)),
            full((34 * _H, _H)), full((1, 1)),
        ],
        out_specs=[
            pl.BlockSpec((_NB, 128, _H), lambda b: (b, 0, 0)),
            pl.BlockSpec((_NB, 128, _H), lambda b: (b, 0, 0)),
            pl.BlockSpec((1, _NB, 128), lambda b: (b, 0, 0)),
        ],
        out_shape=[
            jax.ShapeDtypeStruct((_B, 128, _H), _F32),
            jax.ShapeDtypeStruct((_B, 128, _H), _F32),
            jax.ShapeDtypeStruct((_B // _NB, _NB, 128), jnp.int32),
        ],
        compiler_params=pltpu.CompilerParams(
            dimension_semantics=("parallel",)),
    )(xp, codebook, cb16, cbt, cbt16, w0b, enclo, enchi, encb,
      decw, decb, d5, db5)

    return (y.reshape(_B, 1, _L), ze.reshape(_B * 128, _H),
            ids.reshape(_B * 128))


# speedup vs baseline: 4.5251x; 1.0860x over previous
import jax
import jax.numpy as jnp
import numpy as np
from jax.experimental import pallas as pl
from jax.experimental.pallas import tpu as pltpu

_H = 64
_NL = 6
_CB = 1024
_B = 16
_L = 8192
_F32 = jnp.float32
_BF16 = jnp.bfloat16


def _dotb(a, b):
    return jax.lax.dot_general(
        a, b, (((1,), (0,)), ((), ())),
        preferred_element_type=_F32)


_NB = 8
_R = 128 * _NB


def _row_iota():
    return jax.lax.broadcasted_iota(jnp.int32, (_R, 1), 0)


def _sd(a):
    s = jnp.concatenate([jnp.zeros((1, a.shape[1]), a.dtype), a[:-1]], axis=0)
    return jnp.where((_row_iota() & 127) != 0, s, jnp.zeros_like(s))


def _su(a):
    s = jnp.concatenate([a[1:], jnp.zeros((1, a.shape[1]), a.dtype)], axis=0)
    return jnp.where((_row_iota() & 127) != 127, s, jnp.zeros_like(s))


def _wrap(z):
    return jnp.concatenate([_sd(z[:, -_H:]), z, _su(z[:, :_H])], axis=1)


def _body(x_ref, cb_ref, cb16_ref, cbt_ref, cbt16_ref, w0_ref,
          enclo_ref, enchi_ref, encb_ref, decw_ref, decb_ref,
          d5_ref, db5_ref, y_ref, ze_ref, ids_ref):
    x0 = x_ref[:].reshape(_R, _H).astype(_BF16)
    x0x = jnp.concatenate([_sd(x0[:, -1:]), x0, _su(x0[:, :1])], axis=1)
    z = _dotb(x0x, w0_ref[:])
    b0 = encb_ref[0]
    z = jnp.maximum(z + jnp.concatenate([b0] * 32, axis=1), 0.0)

    for i in range(1, _NL):
        p = 1 << (6 - i)
        zx = _wrap(z.astype(_BF16))
        outs = []
        for q in range(p // 2):
            lo = zx[:, 128 * q:128 * q + 128]
            hi = zx[:, 128 * q + 128:128 * q + 256]
            outs.append(_dotb(lo, enclo_ref[i - 1])
                        + _dotb(hi, enchi_ref[i - 1]))
        acc = outs[0] if len(outs) == 1 else jnp.concatenate(outs, axis=1)
        bt = (encb_ref[i] if p == 2
              else jnp.concatenate([encb_ref[i]] * (p // 2), axis=1))
        z = jnp.maximum(acc + bt, 0.0)

    cbt = cbt_ref[:]
    scores = _dotb(z.astype(_BF16), cbt16_ref[:])
    cbn = jnp.sum(cbt * cbt, axis=0, keepdims=True)
    z2 = jnp.sum(z * z, axis=1, keepdims=True)
    dist = z2 - 2.0 * scores + cbn
    dmin = jnp.min(dist, axis=1, keepdims=True)
    iota = jax.lax.broadcasted_iota(jnp.int32, (_R, _CB), 1)
    ids = jnp.min(jnp.where(dist <= dmin, iota, _CB), axis=1)
    onehot = (iota == ids[:, None]).astype(_BF16)
    e = _dotb(onehot, cb16_ref[:])

    ze_ref[:] = z.reshape(_NB, 128, _H)
    ids_ref[0] = ids.astype(jnp.int32).reshape(_NB, 128)

    y = e
    for i in range(_NL - 1):
        p = 1 << i
        yx = _wrap(y.astype(_BF16))
        outs = []
        for q in range(p):
            xm = yx[:, 64 * q:64 * q + 64]
            xc = yx[:, 64 * q + 64:64 * q + 128]
            xq = yx[:, 64 * q + 128:64 * q + 192]
            outs.append(_dotb(xc, decw_ref[i, 1]) + _dotb(xm, decw_ref[i, 3]))
            outs.append(_dotb(xc, decw_ref[i, 2]) + _dotb(xq, decw_ref[i, 0]))
        acc = jnp.concatenate(outs, axis=1)
        bt = jnp.concatenate([decb_ref[i]] * (2 * p), axis=1)
        y = jnp.maximum(acc + bt, 0.0)

    y5 = _wrap(y.astype(_BF16))
    yout = _dotb(y5, d5_ref[:]) + db5_ref[:]
    y_ref[:] = yout.reshape(_NB, 128, _H)


def kernel(x, codebook, enc_w0, enc_w1, enc_w2, enc_w3, enc_w4, enc_w5,
           enc_b0, enc_b1, enc_b2, enc_b3, enc_b4, enc_b5,
           dec_w0, dec_w1, dec_w2, dec_w3, dec_w4, dec_w5,
           dec_b0, dec_b1, dec_b2, dec_b3, dec_b4, dec_b5):
    enc_ws = [enc_w0, enc_w1, enc_w2, enc_w3, enc_w4, enc_w5]
    enc_bs = [enc_b0, enc_b1, enc_b2, enc_b3, enc_b4, enc_b5]
    dec_ws = [dec_w0, dec_w1, dec_w2, dec_w3, dec_w4, dec_w5]
    dec_bs = [dec_b0, dec_b1, dec_b2, dec_b3, dec_b4, dec_b5]

    w0t = enc_ws[0][:, 0, :].T
    pat0 = np.zeros((4, 66, 32, 1), np.float32)
    for q_ in range(32):
        for k_ in range(4):
            pat0[k_, 2 * q_ + k_, q_, 0] = 1.0
    w0b = sum(pat0[k_] * w0t[k_][None, None, :] for k_ in range(4))
    w0b = w0b.reshape(66, 32 * _H).astype(_BF16)

    enclo = jnp.stack(
        [jnp.concatenate([enc_ws[i][:, :, 0].T, enc_ws[i][:, :, 1].T], axis=0)
         for i in range(1, _NL)]).astype(_BF16)
    enchi = jnp.stack(
        [jnp.concatenate([enc_ws[i][:, :, 2].T, enc_ws[i][:, :, 3].T], axis=0)
         for i in range(1, _NL)]).astype(_BF16)
    encb = jnp.stack([b[None, :] for b in enc_bs])
    decw = jnp.stack([jnp.stack([dec_ws[i][:, :, k] for k in range(4)])
                      for i in range(_NL - 1)]).astype(_BF16)
    decb = jnp.stack([b[None, :] for b in dec_bs[:-1]])

    d5n = dec_ws[-1][:, 0, :]
    terms = ((1, 0, 1), (0, 0, 3), (1, 1, 2), (2, 1, 0))
    pat5 = np.zeros((4, 34, 1, _H), np.float32)
    for t_, (j_off, s_off, k_) in enumerate(terms):
        for p_ in range(32):
            pat5[t_, p_ + j_off, 0, 2 * p_ + s_off] = 1.0
    w5 = sum(pat5[t_] * d5n[:, k_][None, :, None]
             for t_, (j_off, s_off, k_) in enumerate(terms))
    d5 = w5.reshape(34 * _H, _H).astype(_BF16)
    db5 = dec_bs[-1][None, :]

    cbt = codebook.T
    cbt16 = cbt.astype(_BF16)
    cb16 = codebook.astype(_BF16)
    xp = x.reshape(_B, 128, _H)

    def full(shape):
        nd = len(shape)
        return pl.BlockSpec(shape, lambda b, _n=nd: (0,) * _n)

    y, ze, ids = pl.pallas_call(
        _body,
        grid=(_B // _NB,),
        in_specs=[
            pl.BlockSpec((_NB, 128, _H), lambda b: (b, 0, 0)),
            full((_CB, _H)), full((_CB, _H)), full((_H, _CB)), full((_H, _CB)),
            full((66, 32 * _H)),
            full((5, 2 * _H, _H)), full((5, 2 * _H, _H)), full((6, 1, _H)),
            full((5, 4, _H, _H)), full((5, 1, _H)),
            full((34 * _H, _H)), full((1, 1)),
        ],
        out_specs=[
            pl.BlockSpec((_NB, 128, _H), lambda b: (b, 0, 0)),
            pl.BlockSpec((_NB, 128, _H), lambda b: (b, 0, 0)),
            pl.BlockSpec((1, _NB, 128), lambda b: (b, 0, 0)),
        ],
        out_shape=[
            jax.ShapeDtypeStruct((_B, 128, _H), _F32),
            jax.ShapeDtypeStruct((_B, 128, _H), _F32),
            jax.ShapeDtypeStruct((_B // _NB, _NB, 128), jnp.int32),
        ],
        compiler_params=pltpu.CompilerParams(
            dimension_semantics=("parallel",)),
    )(xp, codebook, cb16, cbt, cbt16, w0b, enclo, enchi, encb,
      decw, decb, d5, db5)

    return (y.reshape(_B, 1, _L), ze.reshape(_B * 128, _H),
            ids.reshape(_B * 128))
